# Initial kernel scaffold; baseline (speedup 1.0000x reference)
import jax
import jax.numpy as jnp
from jax.experimental import pallas as pl


def kernel(x, edge_index, W1, a1_src, a1_dst, b1, W2, a2_src, a2_dst, b2, W3, a3_src, a3_dst, b3, W4, a4_src, a4_dst, b4, W_lin, b_lin):
    raise NotImplementedError("write your pallas kernel here")


# speedup vs baseline: 43.7401x; 43.7401x over previous
import functools

import jax
import jax.numpy as jnp
from jax import lax
from jax.experimental import pallas as pl
from jax.experimental.pallas import tpu as pltpu
from jax.experimental.pallas import tpu_sc as plsc

N = 10000
E = 320000
F_IN = 128
H = 4
C = 16
D = H * C
NCLS = 40

NC = 2
NS = 16
NW = NC * NS
EW = E // NW
K = 80
NCHUNK = EW // K
NPAD = 10240

_f32 = jnp.float32
_i32 = jnp.int32
_HIGH = jax.lax.Precision.HIGHEST


def _lrelu(x):
    return jnp.maximum(x, 0.0) + 0.2 * jnp.minimum(x, 0.0)


def _elu(x):
    return jnp.where(x > 0, x, jnp.exp(jnp.minimum(x, 0.0)) - 1.0)


def _head_sel():
    rows = lax.broadcasted_iota(_i32, (D, H), 0) // C
    cols = lax.broadcasted_iota(_i32, (D, H), 1)
    return (rows == cols).astype(_f32)


GB = 1000
GN = N // GB

_PARAMS_TC = pltpu.CompilerParams(vmem_limit_bytes=50 * 2**20)


def _attn_outputs(i, h, asrc, adst, as_ref, ad_ref, ms_ref):
    sel = _head_sel()
    as_ = jnp.dot(h * asrc, sel, preferred_element_type=_f32, precision=_HIGH)
    ad_ = jnp.dot(h * adst, sel, preferred_element_type=_f32, precision=_HIGH)
    as_ref[...] = as_
    ad_ref[...] = ad_
    bm = jnp.concatenate(
        [as_.max(axis=0, keepdims=True), jnp.full((1, 12), -1e30, _f32)],
        axis=1)

    @pl.when(i == 0)
    def _():
        ms_ref[...] = bm

    @pl.when(i > 0)
    def _():
        ms_ref[...] = jnp.maximum(ms_ref[...], bm)


def _prep1_body(x_ref, w_ref, asrc_ref, adst_ref, h_ref, as_ref, ad_ref,
                ms_ref):
    i = pl.program_id(0)
    h = jnp.dot(x_ref[...], w_ref[...], preferred_element_type=_f32,
                precision=_HIGH)
    h_ref[...] = h
    _attn_outputs(i, h, asrc_ref[...], adst_ref[...], as_ref, ad_ref, ms_ref)


def _combine(outsc_ref, hprev_ref, ss_ref, b_ref):
    rep = _head_sel().T
    sweights = jnp.dot(ss_ref[...], rep, preferred_element_type=_f32,
                       precision=_HIGH)
    xin = (outsc_ref[0] + outsc_ref[1] + sweights * hprev_ref[...]
           + b_ref[...])
    return _elu(xin)


def _postprep_body(outsc_ref, hprev_ref, ss_ref, b_ref, w_ref, asrc_ref,
                   adst_ref, h_ref, as_ref, ad_ref, ms_ref):
    i = pl.program_id(0)
    x = _combine(outsc_ref, hprev_ref, ss_ref, b_ref)
    h = jnp.dot(x, w_ref[...], preferred_element_type=_f32, precision=_HIGH)
    h_ref[...] = h
    _attn_outputs(i, h, asrc_ref[...], adst_ref[...], as_ref, ad_ref, ms_ref)


def _mid_body(den_ref, asf_ref, adf_ref, msf_ref, r_ref, ss_ref):
    den = jnp.sum(den_ref[...], axis=0)
    asf = asf_ref[...]
    adf = adf_ref[...]
    msf = msf_ref[...]
    ps = jnp.exp(_lrelu(asf + adf) - _lrelu(msf + adf))
    r = 1.0 / (den + ps)
    r_ref[...] = r
    ss_ref[...] = ps * r


def _final_body(outsc_ref, hprev_ref, ss_ref, b_ref, wlin_ref, blin_ref,
                out_ref, acc_ref):
    i = pl.program_id(0)
    x = _combine(outsc_ref, hprev_ref, ss_ref, b_ref)
    bsum = jnp.sum(x, axis=0, keepdims=True)

    @pl.when(i == 0)
    def _():
        acc_ref[...] = bsum

    @pl.when(i > 0)
    def _():
        acc_ref[...] = acc_ref[...] + bsum

    @pl.when(i == GN - 1)
    def _():
        pooled = acc_ref[...] * (1.0 / N)
        logits = jnp.dot(pooled, wlin_ref[...], preferred_element_type=_f32,
                         precision=_HIGH) + blin_ref[...]
        m = jnp.max(logits, axis=1, keepdims=True)
        lse = m + jnp.log(jnp.sum(jnp.exp(logits - m), axis=1,
                                  keepdims=True))
        out_ref[...] = logits - lse


def _rows(shape):
    return pl.BlockSpec(shape, lambda i: (i,) + (0,) * (len(shape) - 1))


def _bcast(shape):
    return pl.BlockSpec(shape, lambda i: (0,) * len(shape))


_prep1 = pl.pallas_call(
    _prep1_body,
    grid=(GN,),
    in_specs=[_rows((GB, F_IN)), _bcast((F_IN, D)), _bcast((1, D)),
              _bcast((1, D))],
    out_specs=(_rows((GB, D)), _rows((GB, H)), _rows((GB, H)),
               _bcast((1, 16))),
    out_shape=(
        jax.ShapeDtypeStruct((N, D), _f32),
        jax.ShapeDtypeStruct((N, H), _f32),
        jax.ShapeDtypeStruct((N, H), _f32),
        jax.ShapeDtypeStruct((1, 16), _f32),
    ),
    compiler_params=_PARAMS_TC,
)

_postprep = pl.pallas_call(
    _postprep_body,
    grid=(GN,),
    in_specs=[pl.BlockSpec((NC, GB, D), lambda i: (0, i, 0)),
              _rows((GB, D)), _rows((GB, H)), _bcast((1, D)),
              _bcast((D, D)), _bcast((1, D)), _bcast((1, D))],
    out_specs=(_rows((GB, D)), _rows((GB, H)), _rows((GB, H)),
               _bcast((1, 16))),
    out_shape=(
        jax.ShapeDtypeStruct((N, D), _f32),
        jax.ShapeDtypeStruct((N, H), _f32),
        jax.ShapeDtypeStruct((N, H), _f32),
        jax.ShapeDtypeStruct((1, 16), _f32),
    ),
    compiler_params=_PARAMS_TC,
)

_mid = pl.pallas_call(
    _mid_body,
    out_shape=(
        jax.ShapeDtypeStruct((NPAD * H,), _f32),
        jax.ShapeDtypeStruct((NPAD * H,), _f32),
    ),
    compiler_params=_PARAMS_TC,
)

_final = pl.pallas_call(
    _final_body,
    grid=(GN,),
    in_specs=[pl.BlockSpec((NC, GB, D), lambda i: (0, i, 0)),
              _rows((GB, D)), _rows((GB, H)), _bcast((1, D)),
              _bcast((D, NCLS)), _bcast((1, NCLS))],
    out_specs=_bcast((1, NCLS)),
    out_shape=jax.ShapeDtypeStruct((1, NCLS), _f32),
    scratch_shapes=[pltpu.VMEM((1, D), _f32)],
    compiler_params=_PARAMS_TC,
)


def _sc_pass1_body(src_hbm, dst_hbm, as_hbm, ad_hbm, ms_hbm, z4_hbm,
                   den_out, p_out, asv, adv, msv, srcv, dstv, pbuf, denv):
    c = lax.axis_index("c")
    s = lax.axis_index("s")
    wid = c * NS + s
    pltpu.sync_copy(as_hbm, asv)
    pltpu.sync_copy(ad_hbm, adv)
    pltpu.sync_copy(ms_hbm, msv)
    pltpu.sync_copy(z4_hbm, denv)

    lanes16 = lax.broadcasted_iota(_i32, (16,), 0)
    ms16 = msv[...]
    mvec = [ms16[h] + jnp.zeros((16,), _f32) for h in range(H)]

    def chunk(i, carry):
        base = wid * EW + i * K
        pltpu.sync_copy(src_hbm.at[pl.ds(base, K)], srcv)
        pltpu.sync_copy(dst_hbm.at[pl.ds(base, K)], dstv)
        for g in range(K // 16):
            sv = srcv[pl.ds(g * 16, 16)] * H
            dv = dstv[pl.ds(g * 16, 16)] * H
            kvec = lanes16 + g * 16
            for h in range(H):
                a1 = plsc.load_gather(asv, [sv + h])
                a2 = plsc.load_gather(adv, [dv + h])
                e = _lrelu(a1 + a2)
                bd = _lrelu(mvec[h] + a2)
                p = jnp.exp(e - bd)
                plsc.store_scatter(pbuf, [kvec, jnp.full((16,), h, _i32)], p)
                plsc.addupdate_scatter(denv, [dv + h], p)
        pltpu.sync_copy(pbuf, p_out.at[pl.ds(base, K)])
        return carry

    lax.fori_loop(0, NCHUNK, chunk, 0)
    pltpu.sync_copy(denv, den_out.at[wid])


def _sc_pass2_body(src_hbm, dst_hbm, p_hbm, r_hbm, h_hbm, z64_hbm,
                   out_hbm, rv, srcv, dstv, pv, wbuf, hrows, out_sh, sem):
    c = lax.axis_index("c")
    s = lax.axis_index("s")
    wid = c * NS + s
    pltpu.sync_copy(r_hbm, rv)
    rows = NPAD // NS
    pltpu.sync_copy(z64_hbm.at[pl.ds(s * rows, rows)],
                    out_sh.at[pl.ds(s * rows, rows)])
    plsc.subcore_barrier()

    lanes16 = lax.broadcasted_iota(_i32, (16,), 0)

    def chunk(i, carry):
        base = wid * EW + i * K
        pltpu.sync_copy(src_hbm.at[pl.ds(base, K)], srcv)
        pltpu.sync_copy(dst_hbm.at[pl.ds(base, K)], dstv)
        pltpu.sync_copy(p_hbm.at[pl.ds(base, K)], pv)
        pltpu.async_copy(h_hbm.at[srcv], hrows, sem).wait()
        for g in range(K // 16):
            dv = dstv[pl.ds(g * 16, 16)] * H
            kvec = lanes16 + g * 16
            for h in range(H):
                hvec = jnp.full((16,), h, _i32)
                pw = plsc.load_gather(pv, [kvec, hvec])
                rw = plsc.load_gather(rv, [dv + h])
                plsc.store_scatter(wbuf, [kvec * H + h], pw * rw)
        for k4 in range(K // 4):
            wv = wbuf[pl.ds(k4 * 16, 16)]
            for j in range(4):
                k = k4 * 4 + j
                for h in range(H):
                    w = wv[j * H + h]
                    hrows[k, pl.ds(h * C, C)] = hrows[k, pl.ds(h * C, C)] * w
        pltpu.sync_copy(hrows, out_sh.at[dstv], add=True)
        return carry

    lax.fori_loop(0, NCHUNK, chunk, 0)
    plsc.subcore_barrier()
    pltpu.sync_copy(out_sh.at[pl.ds(s * rows, rows)],
                    out_hbm.at[c, pl.ds(s * rows, rows)])


@functools.lru_cache(maxsize=1)
def _sc_kernels():
    mesh = plsc.VectorSubcoreMesh(core_axis_name="c", subcore_axis_name="s",
                                  num_cores=NC, num_subcores=NS)
    params = pltpu.CompilerParams(needs_layout_passes=False,
                                  use_tc_tiling_on_sc=False)
    sc_pass1 = pl.kernel(
        _sc_pass1_body,
        out_type=(
            jax.ShapeDtypeStruct((NW, NPAD * H), _f32),
            jax.ShapeDtypeStruct((E, H), _f32),
        ),
        mesh=mesh,
        scratch_types=[
            pltpu.VMEM((N * H,), _f32),
            pltpu.VMEM((N * H,), _f32),
            pltpu.VMEM((16,), _f32),
            pltpu.VMEM((K,), _i32),
            pltpu.VMEM((K,), _i32),
            pltpu.VMEM((K, H), _f32),
            pltpu.VMEM((NPAD * H,), _f32),
        ],
        compiler_params=params,
    )
    sc_pass2 = pl.kernel(
        _sc_pass2_body,
        out_type=jax.ShapeDtypeStruct((NC, NPAD, D), _f32),
        mesh=mesh,
        scratch_types=[
            pltpu.VMEM((NPAD * H,), _f32),
            pltpu.VMEM((K,), _i32),
            pltpu.VMEM((K,), _i32),
            pltpu.VMEM((K, H), _f32),
            pltpu.VMEM((K * H,), _f32),
            pltpu.VMEM((K, D), _f32),
            pltpu.VMEM_SHARED((NPAD, D), _f32),
            pltpu.SemaphoreType.DMA,
        ],
        compiler_params=params,
    )
    return sc_pass1, sc_pass2


def kernel(x, edge_index, W1, a1_src, a1_dst, b1, W2, a2_src, a2_dst, b2,
           W3, a3_src, a3_dst, b3, W4, a4_src, a4_dst, b4, W_lin, b_lin):
    _sc_pass1, _sc_pass2 = _sc_kernels()
    src = edge_index[0]
    dst = edge_index[1]
    z4 = jnp.zeros((NPAD * H,), _f32)
    z64 = jnp.zeros((NPAD, D), _f32)

    layers = [
        (W1, a1_src, a1_dst, b1),
        (W2, a2_src, a2_dst, b2),
        (W3, a3_src, a3_dst, b3),
        (W4, a4_src, a4_dst, b4),
    ]

    W, asrc, adst, b = layers[0]
    h, as2, ad2, ms = _prep1(x, W, asrc.reshape(1, D), adst.reshape(1, D))
    zpad = jnp.zeros((NPAD - N) * H, _f32)

    out = None
    for l in range(4):
        den, p = _sc_pass1(src, dst, as2.reshape(-1), ad2.reshape(-1),
                           ms.reshape(-1), z4)
        asf = jnp.concatenate([as2.reshape(-1), zpad])
        adf = jnp.concatenate([ad2.reshape(-1), zpad])
        msf = jnp.tile(ms[0, :H], NPAD)
        r, ssf = _mid(den, asf, adf, msf)
        outsc = _sc_pass2(src, dst, p, r, h, z64)
        outsc = outsc[:, :N, :]
        ss = ssf[:N * H].reshape(N, H)
        blayer = layers[l][3].reshape(1, D)
        if l == 3:
            out = _final(outsc, h, ss, blayer, W_lin, b_lin.reshape(1, NCLS))
        else:
            Wn, asrcn, adstn, _ = layers[l + 1]
            h, as2, ad2, ms = _postprep(outsc, h, ss, blayer, Wn,
                                        asrcn.reshape(1, D),
                                        adstn.reshape(1, D))
    return out
